# baseline (speedup 1.0000x reference)
import functools

import jax
import jax.numpy as jnp
from jax import lax
from jax.experimental import pallas as pl
from jax.experimental.pallas import tpu as pltpu
from jax.experimental.pallas import tpu_sc as plsc

PRETRAIN_DIM = 128
DIM_LLM = 768
NUM_PREFIX = 1
P_CHUNKS = 2
BM = 4096


def _make_sc_gather(Bc, D):
    info = plsc.get_sparse_core_info()
    NC, NS = info.num_cores, info.num_subcores
    NW = NC * NS
    b_per_w = Bc // NW
    CH = 128
    n_ch = b_per_w // CH
    mesh = plsc.VectorSubcoreMesh(core_axis_name="c", subcore_axis_name="s")

    @functools.partial(
        pl.kernel,
        mesh=mesh,
        out_type=jax.ShapeDtypeStruct((Bc, D), jnp.float32),
        scratch_types=[
            pltpu.VMEM((b_per_w,), jnp.int32),
            pltpu.VMEM((2, CH, D), jnp.float32),
            pltpu.SemaphoreType.DMA,
            pltpu.SemaphoreType.DMA,
        ],
    )
    def gather_kernel(table_hbm, idx_hbm, out_hbm, idx_v, rows_v, gsem, osem):
        wid = lax.axis_index("s") * NC + lax.axis_index("c")
        base = wid * b_per_w
        pltpu.sync_copy(idx_hbm.at[pl.ds(base, b_per_w)], idx_v)
        gathers = [None] * n_ch
        wbacks = [None] * n_ch
        gathers[0] = pltpu.async_copy(
            table_hbm.at[idx_v.at[pl.ds(0, CH)]], rows_v.at[0], gsem
        )
        for j in range(n_ch):
            if j + 1 < n_ch:
                if j >= 1:
                    wbacks[j - 1].wait()
                gathers[j + 1] = pltpu.async_copy(
                    table_hbm.at[idx_v.at[pl.ds((j + 1) * CH, CH)]],
                    rows_v.at[(j + 1) % 2],
                    gsem,
                )
            gathers[j].wait()
            wbacks[j] = pltpu.async_copy(
                rows_v.at[j % 2], out_hbm.at[pl.ds(base + j * CH, CH)], osem
            )
        wbacks[n_ch - 1].wait()
        if n_ch >= 2:
            wbacks[n_ch - 2].wait()

    return gather_kernel


def _matmul_chunk(emb, W, b2, prev_out, row_off, M):
    Bc, K = emb.shape
    N = W.shape[0]
    bm = min(BM, Bc)
    blk_off = row_off // bm

    def _proj(emb_ref, w_ref, b_ref):
        acc = lax.dot_general(
            emb_ref[...],
            w_ref[...],
            dimension_numbers=(((1,), (1,)), ((), ())),
            preferred_element_type=jnp.float32,
        )
        return (acc + b_ref[...])[:, None, :]

    def mm_kernel(emb_ref, w_ref, b_ref, prev_ref, out_ref):
        out_ref[...] = _proj(emb_ref, w_ref, b_ref)

    def mm_kernel_first(emb_ref, w_ref, b_ref, out_ref):
        out_ref[...] = _proj(emb_ref, w_ref, b_ref)

    in_specs = [
        pl.BlockSpec((bm, K), lambda i: (i, 0)),
        pl.BlockSpec((N, K), lambda i: (0, 0)),
        pl.BlockSpec((1, N), lambda i: (0, 0)),
    ]
    out_spec = pl.BlockSpec((bm, 1, N), lambda i, _o=blk_off: (i + _o, 0, 0))
    out_shape = jax.ShapeDtypeStruct((M, 1, N), jnp.float32)
    if prev_out is None:
        return pl.pallas_call(
            mm_kernel_first,
            grid=(Bc // bm,),
            in_specs=in_specs,
            out_specs=out_spec,
            out_shape=out_shape,
        )(emb, W, b2)
    return pl.pallas_call(
        mm_kernel,
        grid=(Bc // bm,),
        in_specs=in_specs + [pl.BlockSpec(memory_space=pltpu.HBM)],
        out_specs=out_spec,
        out_shape=out_shape,
        input_output_aliases={3: 0},
    )(emb, W, b2, prev_out)


def kernel(ent_table, W, b, triple_ids):
    R, L = triple_ids.shape
    B = R * L
    D = ent_table.shape[1]
    b2 = b.reshape(1, -1)
    chunk_sizes = [8192, 24576]
    assert sum(chunk_sizes) == B
    gather_fns = {Bc: _make_sc_gather(Bc, D) for Bc in set(chunk_sizes)}
    out = None
    row_off = 0
    for Bc in chunk_sizes:
        Rc = Bc // L
        r0 = row_off // L
        ids_c = lax.slice(triple_ids, (r0, 0), (r0 + Rc, L)).reshape(-1)
        ids_c = ids_c.astype(jnp.int32)
        emb = gather_fns[Bc](ent_table, ids_c)
        out = _matmul_chunk(emb, W, b2, out, row_off, B)
        row_off += Bc
    return out


# speedup vs baseline: 1.0133x; 1.0133x over previous
import functools

import jax
import jax.numpy as jnp
from jax import lax
from jax.experimental import pallas as pl
from jax.experimental.pallas import tpu as pltpu
from jax.experimental.pallas import tpu_sc as plsc

PRETRAIN_DIM = 128
DIM_LLM = 768
NUM_PREFIX = 1
BM = 4096


def _make_sc_gather(Bc, D):
    info = plsc.get_sparse_core_info()
    NC, NS = info.num_cores, info.num_subcores
    NW = NC * NS
    b_per_w = Bc // NW
    CH = 128
    n_ch = b_per_w // CH
    mesh = plsc.VectorSubcoreMesh(core_axis_name="c", subcore_axis_name="s")

    @functools.partial(
        pl.kernel,
        mesh=mesh,
        out_type=jax.ShapeDtypeStruct((Bc, D), jnp.float32),
        scratch_types=[
            pltpu.VMEM((b_per_w,), jnp.int32),
            pltpu.VMEM((2, CH, D), jnp.float32),
            pltpu.SemaphoreType.DMA,
            pltpu.SemaphoreType.DMA,
        ],
    )
    def gather_kernel(table_hbm, idx_hbm, out_hbm, idx_v, rows_v, gsem, osem):
        wid = lax.axis_index("s") * NC + lax.axis_index("c")
        base = wid * b_per_w
        pltpu.sync_copy(idx_hbm.at[pl.ds(base, b_per_w)], idx_v)
        gathers = [None] * n_ch
        wbacks = [None] * n_ch
        gathers[0] = pltpu.async_copy(
            table_hbm.at[idx_v.at[pl.ds(0, CH)]], rows_v.at[0], gsem
        )
        for j in range(n_ch):
            if j + 1 < n_ch:
                if j >= 1:
                    wbacks[j - 1].wait()
                gathers[j + 1] = pltpu.async_copy(
                    table_hbm.at[idx_v.at[pl.ds((j + 1) * CH, CH)]],
                    rows_v.at[(j + 1) % 2],
                    gsem,
                )
            gathers[j].wait()
            wbacks[j] = pltpu.async_copy(
                rows_v.at[j % 2], out_hbm.at[pl.ds(base + j * CH, CH)], osem
            )
        wbacks[n_ch - 1].wait()
        if n_ch >= 2:
            wbacks[n_ch - 2].wait()

    return gather_kernel


def _matmul_chunk(emb, W, b2, prev_out, row_off, M):
    Bc, K = emb.shape
    N = W.shape[0]
    bm = min(BM, Bc)
    blk_off = row_off // bm

    def _proj(emb_ref, w_ref, b_ref):
        acc = lax.dot_general(
            emb_ref[...],
            w_ref[...],
            dimension_numbers=(((1,), (1,)), ((), ())),
            preferred_element_type=jnp.float32,
        )
        return (acc + b_ref[...])[:, None, :]

    def mm_kernel(emb_ref, w_ref, b_ref, prev_ref, out_ref):
        out_ref[...] = _proj(emb_ref, w_ref, b_ref)

    def mm_kernel_first(emb_ref, w_ref, b_ref, out_ref):
        out_ref[...] = _proj(emb_ref, w_ref, b_ref)

    in_specs = [
        pl.BlockSpec((bm, K), lambda i: (i, 0)),
        pl.BlockSpec((N, K), lambda i: (0, 0)),
        pl.BlockSpec((1, N), lambda i: (0, 0)),
    ]
    out_spec = pl.BlockSpec((bm, 1, N), lambda i, _o=blk_off: (i + _o, 0, 0))
    out_shape = jax.ShapeDtypeStruct((M, 1, N), jnp.float32)
    if prev_out is None:
        return pl.pallas_call(
            mm_kernel_first,
            grid=(Bc // bm,),
            in_specs=in_specs,
            out_specs=out_spec,
            out_shape=out_shape,
        )(emb, W, b2)
    return pl.pallas_call(
        mm_kernel,
        grid=(Bc // bm,),
        in_specs=in_specs + [pl.BlockSpec(memory_space=pltpu.HBM)],
        out_specs=out_spec,
        out_shape=out_shape,
        input_output_aliases={3: 0},
    )(emb, W, b2, prev_out)


def kernel(ent_table, W, b, triple_ids):
    R, L = triple_ids.shape
    B = R * L
    D = ent_table.shape[1]
    b2 = b.reshape(1, -1)
    chunk_sizes = [16384, 16384]
    assert sum(chunk_sizes) == B
    gather_fns = {Bc: _make_sc_gather(Bc, D) for Bc in set(chunk_sizes)}
    out = None
    row_off = 0
    for Bc in chunk_sizes:
        Rc = Bc // L
        r0 = row_off // L
        ids_c = lax.slice(triple_ids, (r0, 0), (r0 + Rc, L)).reshape(-1)
        ids_c = ids_c.astype(jnp.int32)
        emb = gather_fns[Bc](ent_table, ids_c)
        out = _matmul_chunk(emb, W, b2, out, row_off, B)
        row_off += Bc
    return out
